# Initial kernel scaffold; baseline (speedup 1.0000x reference)
import jax
import jax.numpy as jnp
from jax.experimental import pallas as pl


def kernel(Y_value, W_feat, b_feat, gamma, beta, W_hash, b_hash, W_res, b_res):
    raise NotImplementedError("write your pallas kernel here")


# speedup vs baseline: 18.5193x; 18.5193x over previous
import jax
import jax.numpy as jnp
from jax.experimental import pallas as pl
from jax.experimental.pallas import tpu as pltpu

B = 4096
K = 16
LN_EPS = 1e-5
RB = 128
NEG = -3.0


def _feat_kernel(y_ref, wf_ref, bf_ref, gamma_ref, beta_ref, f_ref, n_ref):
    x = jnp.dot(y_ref[...], wf_ref[...].T, preferred_element_type=jnp.float32)
    x = x + bf_ref[...]
    x = x * 0.5 * (1.0 + jax.lax.erf(x * 0.7071067811865476))
    mu = jnp.mean(x, axis=-1, keepdims=True)
    var = jnp.mean((x - mu) ** 2, axis=-1, keepdims=True)
    f = (x - mu) / jnp.sqrt(var + LN_EPS) * gamma_ref[...] + beta_ref[...]
    f_ref[...] = f
    norm = jnp.sqrt(jnp.sum(f * f, axis=-1, keepdims=True))
    n_ref[...] = f / norm


def _graph_kernel(n_blk_ref, n_all_ref, f_all_ref, g_ref, p1_ref):
    i = pl.program_id(0)
    nb = n_blk_ref[...]
    s = jax.lax.dot_general(nb, n_all_ref[...], (((1,), (1,)), ((), ())),
                            preferred_element_type=jnp.float32)
    rows = i * RB + jax.lax.broadcasted_iota(jnp.int32, (RB, B), 0)
    cols = jax.lax.broadcasted_iota(jnp.int32, (RB, B), 1)
    diag = rows == cols
    s = jnp.where(diag, NEG, s)
    w = s
    vals = []
    for _ in range(K):
        m = jnp.max(w, axis=1, keepdims=True)
        vals.append(m)
        w = jnp.where(w == m, NEG, w)
    m0 = vals[0]
    t_k = vals[K - 1]
    z = vals[0] * 0.0
    for v in vals:
        z = z + jnp.exp(v - m0)
    g = jnp.where(s >= t_k, jnp.exp(s - m0) / (2.0 * z), 0.0)
    g = jnp.where(diag, 0.5, g)
    g_ref[...] = g
    p1_ref[...] = jnp.dot(g, f_all_ref[...], preferred_element_type=jnp.float32)


def _head_kernel(g_ref, p1_all_ref, f_blk_ref, wh_ref, bh_ref, wr_ref, br_ref,
                 h_ref):
    p2 = jnp.dot(g_ref[...], p1_all_ref[...], preferred_element_type=jnp.float32)
    hg = jnp.dot(p2, wh_ref[...].T, preferred_element_type=jnp.float32) + bh_ref[...]
    hr = jnp.dot(f_blk_ref[...], wr_ref[...].T, preferred_element_type=jnp.float32) + br_ref[...]
    h_ref[...] = jnp.tanh(0.5 * hg + 0.5 * hr)


def kernel(Y_value, W_feat, b_feat, gamma, beta, W_hash, b_hash, W_res, b_res):
    fdim = W_feat.shape[0]
    bf = b_feat.reshape(1, fdim)
    gam = gamma.reshape(1, fdim)
    bet = beta.reshape(1, fdim)
    bh = b_hash.reshape(1, -1)
    br = b_res.reshape(1, -1)

    f_value, n_value = pl.pallas_call(
        _feat_kernel,
        out_shape=(
            jax.ShapeDtypeStruct((B, fdim), jnp.float32),
            jax.ShapeDtypeStruct((B, fdim), jnp.float32),
        ),
    )(Y_value, W_feat, bf, gam, bet)

    n_panels = B // RB
    g_value, prop1 = pl.pallas_call(
        _graph_kernel,
        grid=(n_panels,),
        in_specs=[
            pl.BlockSpec((RB, fdim), lambda i: (i, 0)),
            pl.BlockSpec((B, fdim), lambda i: (0, 0)),
            pl.BlockSpec((B, fdim), lambda i: (0, 0)),
        ],
        out_specs=(
            pl.BlockSpec((RB, B), lambda i: (i, 0)),
            pl.BlockSpec((RB, fdim), lambda i: (i, 0)),
        ),
        out_shape=(
            jax.ShapeDtypeStruct((B, B), jnp.float32),
            jax.ShapeDtypeStruct((B, fdim), jnp.float32),
        ),
        compiler_params=pltpu.CompilerParams(
            dimension_semantics=("arbitrary",),
        ),
    )(n_value, n_value, f_value)

    h_value = pl.pallas_call(
        _head_kernel,
        grid=(n_panels,),
        in_specs=[
            pl.BlockSpec((RB, B), lambda i: (i, 0)),
            pl.BlockSpec((B, fdim), lambda i: (0, 0)),
            pl.BlockSpec((RB, fdim), lambda i: (i, 0)),
            pl.BlockSpec(W_hash.shape, lambda i: (0, 0)),
            pl.BlockSpec((1, bh.shape[1]), lambda i: (0, 0)),
            pl.BlockSpec(W_res.shape, lambda i: (0, 0)),
            pl.BlockSpec((1, br.shape[1]), lambda i: (0, 0)),
        ],
        out_specs=pl.BlockSpec((RB, W_hash.shape[0]), lambda i: (i, 0)),
        out_shape=jax.ShapeDtypeStruct((B, W_hash.shape[0]), jnp.float32),
        compiler_params=pltpu.CompilerParams(
            dimension_semantics=("arbitrary",),
        ),
    )(g_value, prop1, f_value, W_hash, bh, W_res, br)

    return (f_value, g_value, h_value)
